# baseline (speedup 1.0000x reference)
import functools

import jax
import jax.numpy as jnp
from jax.experimental import pallas as pl

_NA = 3
_NC = 10
_NO = _NA * _NC


def _mm_body(x_ref, w_ref, b_ref, o_ref):
    y = jax.lax.dot_general(
        w_ref[...], x_ref[0],
        dimension_numbers=(((1,), (0,)), ((), ())),
        preferred_element_type=jnp.float32,
    )
    o_ref[0] = y + b_ref[...]


def _head(x, W, b, n_tiles):
    B, C, H, Wd = x.shape
    hw = H * Wd
    t = hw // n_tiles
    xp = x.astype(jnp.bfloat16).reshape(B, C, hw)
    z = pl.pallas_call(
        _mm_body,
        grid=(B, n_tiles),
        in_specs=[
            pl.BlockSpec((1, C, t), lambda bi, ti: (bi, 0, ti)),
            pl.BlockSpec((_NO, C), lambda bi, ti: (0, 0)),
            pl.BlockSpec((_NO, 1), lambda bi, ti: (0, 0)),
        ],
        out_specs=pl.BlockSpec((1, _NO, t), lambda bi, ti: (bi, 0, ti)),
        out_shape=jax.ShapeDtypeStruct((B, _NO, hw), jnp.float32),
    )(xp, W.astype(jnp.bfloat16), b.reshape(_NO, 1))
    z = z.reshape(B, _NA, _NC, H, Wd)
    return jnp.transpose(z, (0, 1, 3, 4, 2))


def kernel(p3, p4, p5, W1, b1, W2, b2, W3, b3):
    o3 = _head(p3, W1, b1, 2)
    o4 = _head(p4, W2, b2, 1)
    o5 = _head(p5, W3, b3, 1)
    return (o3, o4, o5)


# speedup vs baseline: 1.0974x; 1.0974x over previous
import functools

import jax
import jax.numpy as jnp
from jax.experimental import pallas as pl

_NA = 3
_NC = 10
_NO = _NA * _NC


def _mm_body(x_ref, w_ref, b_ref, o_ref):
    y = jax.lax.dot_general(
        w_ref[...], x_ref[0],
        dimension_numbers=(((1,), (0,)), ((), ())),
        preferred_element_type=jnp.float32,
    )
    o_ref[0] = y + b_ref[...]


def _head(x, W, b, n_tiles):
    B, C, H, Wd = x.shape
    hw = H * Wd
    t = hw // n_tiles
    xp = x.reshape(B, C, hw)
    z = pl.pallas_call(
        _mm_body,
        grid=(B, n_tiles),
        in_specs=[
            pl.BlockSpec((1, C, t), lambda bi, ti: (bi, 0, ti)),
            pl.BlockSpec((_NO, C), lambda bi, ti: (0, 0)),
            pl.BlockSpec((_NO, 1), lambda bi, ti: (0, 0)),
        ],
        out_specs=pl.BlockSpec((1, _NO, t), lambda bi, ti: (bi, 0, ti)),
        out_shape=jax.ShapeDtypeStruct((B, _NO, hw), jnp.float32),
    )(xp, W, b.reshape(_NO, 1))
    z = z.reshape(B, _NA, _NC, H, Wd)
    return jnp.transpose(z, (0, 1, 3, 4, 2))


def kernel(p3, p4, p5, W1, b1, W2, b2, W3, b3):
    o3 = _head(p3, W1, b1, 1)
    o4 = _head(p4, W2, b2, 1)
    o5 = _head(p5, W3, b3, 1)
    return (o3, o4, o5)


# speedup vs baseline: 1.1554x; 1.0528x over previous
import functools

import jax
import jax.numpy as jnp
from jax.experimental import pallas as pl

_NA = 3
_NC = 10
_NO = _NA * _NC


def _mm_body(x_ref, w_ref, b_ref, o_ref, *, nb):
    for i in range(nb):
        y = jax.lax.dot_general(
            w_ref[...], x_ref[i],
            dimension_numbers=(((1,), (0,)), ((), ())),
            preferred_element_type=jnp.float32,
        )
        o_ref[i] = y + b_ref[...]


def _head(x, W, b, nb):
    B, C, H, Wd = x.shape
    hw = H * Wd
    xp = x.reshape(B, C, hw)
    z = pl.pallas_call(
        functools.partial(_mm_body, nb=nb),
        grid=(B // nb,),
        in_specs=[
            pl.BlockSpec((nb, C, hw), lambda bi: (bi, 0, 0)),
            pl.BlockSpec((_NO, C), lambda bi: (0, 0)),
            pl.BlockSpec((_NO, 1), lambda bi: (0, 0)),
        ],
        out_specs=pl.BlockSpec((nb, _NO, hw), lambda bi: (bi, 0, 0)),
        out_shape=jax.ShapeDtypeStruct((B, _NO, hw), jnp.float32),
    )(xp, W, b.reshape(_NO, 1))
    z = z.reshape(B, _NA, _NC, H, Wd)
    return jnp.transpose(z, (0, 1, 3, 4, 2))


def kernel(p3, p4, p5, W1, b1, W2, b2, W3, b3):
    o3 = _head(p3, W1, b1, 1)
    o4 = _head(p4, W2, b2, 2)
    o5 = _head(p5, W3, b3, 4)
    return (o3, o4, o5)


# speedup vs baseline: 1.1888x; 1.0289x over previous
import functools

import jax
import jax.numpy as jnp
from jax.experimental import pallas as pl

_NA = 3
_NC = 10
_NO = _NA * _NC


def _mm_body(x_ref, w_ref, b_ref, o_ref, *, nb):
    for i in range(nb):
        y = jax.lax.dot_general(
            w_ref[...], x_ref[i],
            dimension_numbers=(((1,), (0,)), ((), ())),
            preferred_element_type=jnp.float32,
        )
        o_ref[i] = y + b_ref[...]


def _head(x, W, b, nb):
    B, C, H, Wd = x.shape
    hw = H * Wd
    xp = x.reshape(B, C, hw)
    z = pl.pallas_call(
        functools.partial(_mm_body, nb=nb),
        grid=(B // nb,),
        in_specs=[
            pl.BlockSpec((nb, C, hw), lambda bi: (bi, 0, 0)),
            pl.BlockSpec((_NO, C), lambda bi: (0, 0)),
            pl.BlockSpec((_NO, 1), lambda bi: (0, 0)),
        ],
        out_specs=pl.BlockSpec((nb, _NO, hw), lambda bi: (bi, 0, 0)),
        out_shape=jax.ShapeDtypeStruct((B, _NO, hw), jnp.float32),
    )(xp, W, b.reshape(_NO, 1))
    z = z.reshape(B, _NA, _NC, H, Wd)
    return jnp.transpose(z, (0, 1, 3, 4, 2))


def kernel(p3, p4, p5, W1, b1, W2, b2, W3, b3):
    o3 = _head(p3, W1, b1, 2)
    o4 = _head(p4, W2, b2, 4)
    o5 = _head(p5, W3, b3, 8)
    return (o3, o4, o5)
